# Initial kernel scaffold; baseline (speedup 1.0000x reference)
import jax
import jax.numpy as jnp
from jax.experimental import pallas as pl


def kernel(logits):
    raise NotImplementedError("write your pallas kernel here")


# speedup vs baseline: 108.3047x; 108.3047x over previous
import functools

import numpy as np

import jax
import jax.numpy as jnp
from jax import lax
from jax.experimental import pallas as pl
from jax.experimental.pallas import tpu as pltpu
from jax.experimental.pallas import tpu_sc as plsc

TOP_P = 0.9

B = 64
V = 100000
NC, NS, L = 2, 16, 16
NW = NC * NS
ROWS_PER_W = B // NW
NB = 4096
NVREG = V // L
L1_SHIFT = 20
L2_SHIFT = 8
SIGN = np.int32(-(2**31))
LOW31 = np.int32(0x7FFFFFFF)


def _monokey(l):
    b = lax.bitcast_convert_type(l, jnp.int32)
    m = lax.shift_right_arithmetic(b, 31)
    ikey = b ^ jnp.bitwise_and(m, LOW31)
    return ikey ^ SIGN


def _scan_hist(h, tlo_v, base_v):
    nch = NB // L

    def cond(st):
        j, _, found = st
        return jnp.logical_and(jnp.logical_not(found), j < nch - 1)

    def body(st):
        j, carry_v, _ = st
        v = h[pl.ds(j * L, L)]
        tot_v = carry_v + jnp.broadcast_to(jnp.sum(v), (L,))
        found = jnp.any(tot_v >= tlo_v)
        jn = jnp.where(found, j, j + 1)
        cn = jnp.where(found, carry_v, tot_v)
        return jn, cn, found

    j, carry_v, _ = lax.while_loop(cond, body, (jnp.int32(0), base_v, False))
    v = h[pl.ds(j * L, L)]
    cs = plsc.cumsum(v) + carry_v
    cross = cs >= tlo_v
    nbelow = plsc.all_reduce_population_count(jnp.logical_not(cross))
    bin_v = nbelow + j * L
    mbe_v = carry_v + jnp.broadcast_to(jnp.sum(jnp.where(cross, 0.0, v)), (L,))
    return bin_v, mbe_v


def _tile_body(logits_hbm, out_hbm, row_v, h1, h2):
    wid = lax.axis_index("c") * NS + lax.axis_index("s")

    for r in range(ROWS_PER_W):
        row = wid * ROWS_PER_W + r
        pltpu.sync_copy(logits_hbm.at[row], row_v)

        zero = jnp.zeros((L,), jnp.float32)

        def zbody(i, _):
            h1[pl.ds(i * L, L)] = zero
            h2[pl.ds(i * L, L)] = zero
            return 0

        lax.fori_loop(0, NB // L, zbody, 0)

        def p1(i, _):
            l = row_v[pl.ds(i * L, L)]
            w = jnp.exp(l)
            key = _monokey(l)
            bin1 = lax.shift_right_logical(key, L1_SHIFT)
            plsc.addupdate_scatter(h1, [bin1], w)
            return 0

        lax.fori_loop(0, NVREG, p1, 0, unroll=4)

        def zsum(i, acc):
            return acc + h1[pl.ds(i * L, L)]

        z_acc = lax.fori_loop(0, NB // L, zsum, zero, unroll=4)
        z_v = jnp.broadcast_to(jnp.sum(z_acc), (L,))
        tlo_v = z_v - jnp.float32(TOP_P) * z_v

        b1_v, mbe1_v = _scan_hist(h1, tlo_v, jnp.zeros((L,), jnp.float32))

        def p2(i, _):
            l = row_v[pl.ds(i * L, L)]
            w = jnp.exp(l)
            key = _monokey(l)
            bin1 = lax.shift_right_logical(key, L1_SHIFT)
            bin2 = jnp.bitwise_and(
                lax.shift_right_logical(key, L2_SHIFT), jnp.int32(NB - 1))
            plsc.addupdate_scatter(h2, [bin2], w, mask=bin1 == b1_v)
            return 0

        lax.fori_loop(0, NVREG, p2, 0, unroll=4)

        b2_v, mbe2_v = _scan_hist(h2, tlo_v, mbe1_v)

        s_v = z_v - mbe2_v
        inv_s = jnp.float32(1.0) / s_v
        keyt_v = jnp.bitwise_or(
            lax.shift_left(b1_v, L1_SHIFT), lax.shift_left(b2_v, L2_SHIFT))
        ikeyt_v = keyt_v ^ SIGN

        def p3(i, _):
            l = row_v[pl.ds(i * L, L)]
            w = jnp.exp(l)
            key = _monokey(l)
            keep = (key ^ SIGN) >= ikeyt_v
            row_v[pl.ds(i * L, L)] = jnp.where(keep, w * inv_s, 0.0)
            return 0

        lax.fori_loop(0, NVREG, p3, 0, unroll=4)

        pltpu.sync_copy(row_v, out_hbm.at[row])


_mesh = plsc.VectorSubcoreMesh(core_axis_name="c", subcore_axis_name="s")

_topp = functools.partial(
    pl.kernel,
    out_type=jax.ShapeDtypeStruct((B, V), jnp.float32),
    mesh=_mesh,
    compiler_params=pltpu.CompilerParams(needs_layout_passes=False),
    scratch_types=[
        pltpu.VMEM((V,), jnp.float32),
        pltpu.VMEM((NB,), jnp.float32),
        pltpu.VMEM((NB,), jnp.float32),
    ],
)(_tile_body)


@jax.jit
def kernel(logits):
    return _topp(logits)


# speedup vs baseline: 316.0409x; 2.9181x over previous
import functools

import numpy as np

import jax
import jax.numpy as jnp
from jax import lax
from jax.experimental import pallas as pl
from jax.experimental.pallas import tpu as pltpu
from jax.experimental.pallas import tpu_sc as plsc

TOP_P = 0.9

B = 64
V = 100000
NC, NS, L = 2, 16, 16
NW = NC * NS
ROWS_PER_W = B // NW
NB = 4096
NVREG = V // L
L1_SHIFT = 20
L2_SHIFT = 8
SIGN = np.int32(-(2**31))
LOW31 = np.int32(0x7FFFFFFF)


def _monokey(l):
    b = lax.bitcast_convert_type(l, jnp.int32)
    m = lax.shift_right_arithmetic(b, 31)
    ikey = b ^ jnp.bitwise_and(m, LOW31)
    return ikey ^ SIGN


def _scan_hist(h, tlo_v, base_v):
    nch = NB // L

    def cond(st):
        j, _, found = st
        return jnp.logical_and(jnp.logical_not(found), j < nch - 1)

    def body(st):
        j, carry_v, _ = st
        v = h[pl.ds(j * L, L)]
        tot_v = carry_v + jnp.broadcast_to(jnp.sum(v), (L,))
        found = jnp.any(tot_v >= tlo_v)
        jn = jnp.where(found, j, j + 1)
        cn = jnp.where(found, carry_v, tot_v)
        return jn, cn, found

    j, carry_v, _ = lax.while_loop(cond, body, (jnp.int32(0), base_v, False))
    v = h[pl.ds(j * L, L)]
    cs = plsc.cumsum(v) + carry_v
    cross = cs >= tlo_v
    nbelow = plsc.all_reduce_population_count(jnp.logical_not(cross))
    bin_v = nbelow + j * L
    mbe_v = carry_v + jnp.broadcast_to(jnp.sum(jnp.where(cross, 0.0, v)), (L,))
    return bin_v, mbe_v


def _tile_body(logits_hbm, out_hbm, row_v, h1, h2):
    wid = lax.axis_index("c") * NS + lax.axis_index("s")

    for r in range(ROWS_PER_W):
        row = wid * ROWS_PER_W + r
        pltpu.sync_copy(logits_hbm.at[row], row_v)

        zero = jnp.zeros((L,), jnp.float32)

        @plsc.parallel_loop(0, NB // L, unroll=16)
        def zbody(i):
            h1[pl.ds(i * L, L)] = zero
            h2[pl.ds(i * L, L)] = zero

        @plsc.parallel_loop(0, NVREG, unroll=10)
        def p1(i):
            l = row_v[pl.ds(i * L, L)]
            w = jnp.exp(l)
            key = _monokey(l)
            bin1 = lax.shift_right_logical(key, L1_SHIFT)
            plsc.addupdate_scatter(h1, [bin1], w)

        def zsum(i, accs):
            return tuple(
                acc + h1[pl.ds((4 * i + k) * L, L)] for k, acc in enumerate(accs))

        z_accs = lax.fori_loop(0, NB // (4 * L), zsum, (zero,) * 4, unroll=4)
        z_v = jnp.broadcast_to(jnp.sum(sum(z_accs)), (L,))
        tlo_v = z_v - jnp.float32(TOP_P) * z_v

        b1_v, mbe1_v = _scan_hist(h1, tlo_v, jnp.zeros((L,), jnp.float32))

        @plsc.parallel_loop(0, NVREG, unroll=10)
        def p2(i):
            l = row_v[pl.ds(i * L, L)]
            w = jnp.exp(l)
            key = _monokey(l)
            bin1 = lax.shift_right_logical(key, L1_SHIFT)
            bin2 = jnp.bitwise_and(
                lax.shift_right_logical(key, L2_SHIFT), jnp.int32(NB - 1))
            plsc.addupdate_scatter(h2, [bin2], w, mask=bin1 == b1_v)

        b2_v, mbe2_v = _scan_hist(h2, tlo_v, mbe1_v)

        s_v = z_v - mbe2_v
        inv_s = jnp.float32(1.0) / s_v
        keyt_v = jnp.bitwise_or(
            lax.shift_left(b1_v, L1_SHIFT), lax.shift_left(b2_v, L2_SHIFT))
        ikeyt_v = keyt_v ^ SIGN

        @plsc.parallel_loop(0, NVREG, unroll=10)
        def p3(i):
            l = row_v[pl.ds(i * L, L)]
            w = jnp.exp(l)
            key = _monokey(l)
            keep = (key ^ SIGN) >= ikeyt_v
            row_v[pl.ds(i * L, L)] = jnp.where(keep, w * inv_s, 0.0)

        pltpu.sync_copy(row_v, out_hbm.at[row])


_mesh = plsc.VectorSubcoreMesh(core_axis_name="c", subcore_axis_name="s")

_topp = functools.partial(
    pl.kernel,
    out_type=jax.ShapeDtypeStruct((B, V), jnp.float32),
    mesh=_mesh,
    compiler_params=pltpu.CompilerParams(needs_layout_passes=False),
    scratch_types=[
        pltpu.VMEM((V,), jnp.float32),
        pltpu.VMEM((NB,), jnp.float32),
        pltpu.VMEM((NB,), jnp.float32),
    ],
)(_tile_body)


@jax.jit
def kernel(logits):
    return _topp(logits)
